# Initial kernel scaffold; baseline (speedup 1.0000x reference)
import jax
import jax.numpy as jnp
from jax.experimental import pallas as pl


def kernel(hour, weekday, norm_time, hour_table, weekday_table):
    raise NotImplementedError("write your pallas kernel here")


# speedup vs baseline: 2.7616x; 2.7616x over previous
import functools
import math

import jax
import jax.numpy as jnp
from jax import lax
from jax.experimental import pallas as pl
from jax.experimental.pallas import tpu as pltpu
from jax.experimental.pallas import tpu_sc as plsc

EMBED = 32
OUT_W = 70
LANES = 16
NUM_CORES = 2
NUM_SUBCORES = 16
NUM_WORKERS = NUM_CORES * NUM_SUBCORES
CHUNK = 512
GROUPS = CHUNK // LANES

_C1 = 2.0 * math.pi
_C3 = -((2.0 * math.pi) ** 3) / 6.0
_C5 = ((2.0 * math.pi) ** 5) / 120.0
_C7 = -((2.0 * math.pi) ** 7) / 5040.0
_C9 = ((2.0 * math.pi) ** 9) / 362880.0


def _sin2pi(x):
    u = x - x.astype(jnp.int32).astype(jnp.float32)
    v = u - 0.5
    av = jnp.abs(v)
    m = jnp.minimum(av, 0.5 - av)
    w = m * jnp.sign(v)
    z = w * w
    p = ((((_C9 * z + _C7) * z + _C5) * z + _C3) * z + _C1)
    return -(p * w)


def _body(hour_hbm, wd_hbm, nt_hbm, htab_hbm, wtab_hbm, out_hbm,
          htab_v, wtab_v, hour_v, wd_v, nt_v, out_v, *, per_worker):
    wid = lax.axis_index("s") * NUM_CORES + lax.axis_index("c")
    base = wid * per_worker
    n_chunks = per_worker // CHUNK

    pltpu.sync_copy(htab_hbm, htab_v)
    pltpu.sync_copy(wtab_hbm, wtab_v)

    iota70 = lax.iota(jnp.int32, LANES) * OUT_W

    def group_body(g, _):
        sl = pl.ds(g * LANES, LANES)
        h = hour_v[sl]
        w = wd_v[sl]
        t = nt_v[sl]
        h32 = h * EMBED
        w32 = w * EMBED
        obase = iota70 + g * (LANES * OUT_W)
        for d in range(EMBED):
            hv = plsc.load_gather(htab_v, [h32 + d])
            plsc.store_scatter(out_v, [obase + d], hv)
        for d in range(EMBED):
            wv = plsc.load_gather(wtab_v, [w32 + d])
            plsc.store_scatter(out_v, [obase + (EMBED + d)], wv)
        s1 = _sin2pi(t)
        c1 = _sin2pi(t + 0.25)
        s2 = 2.0 * s1 * c1
        c2 = 1.0 - 2.0 * s1 * s1
        s4 = 2.0 * s2 * c2
        c4 = 1.0 - 2.0 * s2 * s2
        for k, val in enumerate((s1, c1, s2, c2, s4, c4)):
            plsc.store_scatter(out_v, [obase + (2 * EMBED + k)], val)
        return 0

    def chunk_body(c, _):
        row0 = base + c * CHUNK
        pltpu.sync_copy(hour_hbm.at[pl.ds(row0, CHUNK)], hour_v)
        pltpu.sync_copy(wd_hbm.at[pl.ds(row0, CHUNK)], wd_v)
        pltpu.sync_copy(nt_hbm.at[pl.ds(row0, CHUNK)], nt_v)
        lax.fori_loop(0, GROUPS, group_body, 0)
        pltpu.sync_copy(out_v, out_hbm.at[pl.ds(row0 * OUT_W, CHUNK * OUT_W)])
        return 0

    lax.fori_loop(0, n_chunks, chunk_body, 0)


@functools.cache
def _build(n_tokens, n_hour, n_wd):
    per_worker = n_tokens // NUM_WORKERS
    mesh = plsc.VectorSubcoreMesh(core_axis_name="c", subcore_axis_name="s")
    return pl.kernel(
        functools.partial(_body, per_worker=per_worker),
        out_type=jax.ShapeDtypeStruct((n_tokens * OUT_W,), jnp.float32),
        mesh=mesh,
        compiler_params=pltpu.CompilerParams(needs_layout_passes=False),
        scratch_types=[
            pltpu.VMEM((n_hour * EMBED,), jnp.float32),
            pltpu.VMEM((n_wd * EMBED,), jnp.float32),
            pltpu.VMEM((CHUNK,), jnp.int32),
            pltpu.VMEM((CHUNK,), jnp.int32),
            pltpu.VMEM((CHUNK,), jnp.float32),
            pltpu.VMEM((CHUNK * OUT_W,), jnp.float32),
        ],
    )


def kernel(hour, weekday, norm_time, hour_table, weekday_table):
    b, l = hour.shape
    n_tokens = b * l
    assert n_tokens % (NUM_WORKERS * CHUNK) == 0
    fn = _build(n_tokens, hour_table.shape[0], weekday_table.shape[0])
    out = fn(
        hour.reshape(-1).astype(jnp.int32),
        weekday.reshape(-1).astype(jnp.int32),
        norm_time.reshape(-1),
        hour_table.reshape(-1),
        weekday_table.reshape(-1),
    )
    return out.reshape(b, l, OUT_W)
